# Initial kernel scaffold; baseline (speedup 1.0000x reference)
import jax
import jax.numpy as jnp
from jax.experimental import pallas as pl


def kernel(x, edge_index, W_hidden, W_mean, W_logstd):
    raise NotImplementedError("write your pallas kernel here")


# speedup vs baseline: 8.5721x; 8.5721x over previous
import jax
import jax.numpy as jnp
from jax import lax
from jax.experimental import pallas as pl
from jax.experimental.pallas import tpu as pltpu
from jax.experimental.pallas import tpu_sc as plsc

N = 10000
E = 320000
D = 128
H = 32
C = 16

NC = 2
NS = 16
NW = NC * NS
B = 128
KCH = 80
EW = KCH * B
EPAD = NW * EW
NP = 10112
RT = NP // NS
W = 128


def _sc_mesh():
    return plsc.VectorSubcoreMesh(core_axis_name="c", subcore_axis_name="s")


def _deg_body(dst_hbm, ones_hbm, zeros_hbm, out_hbm, dst_v, ones_v, acc_sh, sem):
    cid = lax.axis_index("c")
    sid = lax.axis_index("s")
    wid = sid * NC + cid

    pltpu.sync_copy(zeros_hbm.at[pl.ds(sid * RT, RT)],
                    acc_sh.at[pl.ds(sid * RT, RT)])
    pltpu.sync_copy(dst_hbm.at[wid], dst_v)
    pltpu.sync_copy(ones_hbm, ones_v)
    plsc.subcore_barrier()

    def body(j, carry):
        pltpu.sync_copy(ones_v, acc_sh.at[dst_v.at[j]], add=True)
        return carry

    lax.fori_loop(0, KCH, body, 0)
    plsc.subcore_barrier()
    pltpu.sync_copy(acc_sh.at[pl.ds(sid * RT, RT)],
                    out_hbm.at[cid, pl.ds(sid * RT, RT)])


def _sc_deg(dst_p, onesW, zerosW):
    call = pl.kernel(
        _deg_body,
        out_type=jax.ShapeDtypeStruct((NC, NP, W), jnp.float32),
        mesh=_sc_mesh(),
        scratch_types=[
            pltpu.VMEM((KCH, B), jnp.int32),
            pltpu.VMEM((B, W), jnp.float32),
            pltpu.VMEM_SHARED((NP, W), jnp.float32),
            pltpu.SemaphoreType.DMA,
        ],
    )
    return call(dst_p, onesW, zerosW)


def _spmm_body(src_hbm, dst_hbm, feat_hbm, zeros_hbm, out_hbm,
               src_v, dst_v, rows_v, acc_sh, sem):
    cid = lax.axis_index("c")
    sid = lax.axis_index("s")
    wid = sid * NC + cid

    pltpu.sync_copy(zeros_hbm.at[pl.ds(sid * RT, RT)],
                    acc_sh.at[pl.ds(sid * RT, RT)])
    pltpu.sync_copy(src_hbm.at[wid], src_v)
    pltpu.sync_copy(dst_hbm.at[wid], dst_v)
    plsc.subcore_barrier()

    def body(j, carry):
        pltpu.async_copy(feat_hbm.at[src_v.at[j]], rows_v, sem).wait()
        pltpu.sync_copy(rows_v, acc_sh.at[dst_v.at[j]], add=True)
        return carry

    lax.fori_loop(0, KCH, body, 0)
    plsc.subcore_barrier()
    pltpu.sync_copy(acc_sh.at[pl.ds(sid * RT, RT)],
                    out_hbm.at[cid, pl.ds(sid * RT, RT)])


def _sc_spmm(src_p, dst_p, feat, zerosW):
    call = pl.kernel(
        _spmm_body,
        out_type=jax.ShapeDtypeStruct((NC, NP, W), jnp.float32),
        mesh=_sc_mesh(),
        scratch_types=[
            pltpu.VMEM((KCH, B), jnp.int32),
            pltpu.VMEM((KCH, B), jnp.int32),
            pltpu.VMEM((B, W), jnp.float32),
            pltpu.VMEM_SHARED((NP, W), jnp.float32),
            pltpu.SemaphoreType.DMA,
        ],
    )
    return call(src_p, dst_p, feat, zerosW)


RBLK = 1000


def _prep0_body(x_ref, w_ref, h_ref):
    h_ref[...] = jnp.dot(x_ref[...], w_ref[...],
                         preferred_element_type=jnp.float32)


def _prep1_body(deg_ref, h_ref, h1_ref, dinv_ref):
    deg = deg_ref[0, :, 0:1] + deg_ref[1, :, 0:1] + 1.0
    dinv = lax.rsqrt(jnp.maximum(deg, 1.0))
    dinvH = jnp.broadcast_to(dinv, (RBLK, H))
    h1 = dinvH * h_ref[...]
    h1_ref[...] = jnp.concatenate(
        [h1, jnp.zeros((RBLK, W - H), jnp.float32)], axis=1)
    dinv_ref[...] = dinvH


def _prep2_body(acc_ref, h1_ref, dinv_ref, wcat_ref, g1_ref):
    s = acc_ref[0, :, :H] + acc_ref[1, :, :H] + h1_ref[:, :H]
    hidden = jnp.maximum(dinv_ref[...] * s, 0.0)
    g = jnp.dot(hidden, wcat_ref[...], preferred_element_type=jnp.float32)
    g1_ref[...] = jnp.concatenate(
        [dinv_ref[...] * g, jnp.zeros((RBLK, W - H), jnp.float32)], axis=1)


def _encode_body(acc_ref, g1_ref, dinv_ref, eps_ref, enc_ref):
    out2 = dinv_ref[...] * (acc_ref[0, :, :H] + acc_ref[1, :, :H]
                            + g1_ref[:, :H])
    means = out2[:, :C]
    log_std = out2[:, C:]
    enc_ref[...] = means + jnp.exp(log_std) * eps_ref[...]


def _tc_prep0(x, W_hidden):
    grid = (N // RBLK,)
    return pl.pallas_call(
        _prep0_body,
        grid=grid,
        in_specs=[
            pl.BlockSpec((RBLK, D), lambda i: (i, 0)),
            pl.BlockSpec((D, H), lambda i: (0, 0)),
        ],
        out_specs=pl.BlockSpec((RBLK, H), lambda i: (i, 0)),
        out_shape=jax.ShapeDtypeStruct((N, H), jnp.float32),
    )(x, W_hidden)


def _tc_prep1(deg_parts, h):
    grid = (N // RBLK,)
    return pl.pallas_call(
        _prep1_body,
        grid=grid,
        in_specs=[
            pl.BlockSpec((NC, RBLK, W), lambda i: (0, i, 0)),
            pl.BlockSpec((RBLK, H), lambda i: (i, 0)),
        ],
        out_specs=[
            pl.BlockSpec((RBLK, W), lambda i: (i, 0)),
            pl.BlockSpec((RBLK, H), lambda i: (i, 0)),
        ],
        out_shape=[
            jax.ShapeDtypeStruct((N, W), jnp.float32),
            jax.ShapeDtypeStruct((N, H), jnp.float32),
        ],
    )(deg_parts, h)


def _tc_prep2(acc_parts, h1p, dinv, Wcat):
    grid = (N // RBLK,)
    return pl.pallas_call(
        _prep2_body,
        grid=grid,
        in_specs=[
            pl.BlockSpec((NC, RBLK, W), lambda i: (0, i, 0)),
            pl.BlockSpec((RBLK, W), lambda i: (i, 0)),
            pl.BlockSpec((RBLK, H), lambda i: (i, 0)),
            pl.BlockSpec((H, H), lambda i: (0, 0)),
        ],
        out_specs=pl.BlockSpec((RBLK, W), lambda i: (i, 0)),
        out_shape=jax.ShapeDtypeStruct((N, W), jnp.float32),
    )(acc_parts, h1p, dinv, Wcat)


def _tc_encode(acc_parts, g1p, dinv, eps):
    grid = (N // RBLK,)
    return pl.pallas_call(
        _encode_body,
        grid=grid,
        in_specs=[
            pl.BlockSpec((NC, RBLK, W), lambda i: (0, i, 0)),
            pl.BlockSpec((RBLK, W), lambda i: (i, 0)),
            pl.BlockSpec((RBLK, H), lambda i: (i, 0)),
            pl.BlockSpec((RBLK, C), lambda i: (i, 0)),
        ],
        out_specs=pl.BlockSpec((RBLK, C), lambda i: (i, 0)),
        out_shape=jax.ShapeDtypeStruct((N, C), jnp.float32),
    )(acc_parts, g1p, dinv, eps)


DM = 512
DN = 2048


def _decode_body(a_ref, b_ref, out_ref):
    z = lax.dot_general(a_ref[...], b_ref[...],
                        (((1,), (1,)), ((), ())),
                        preferred_element_type=jnp.float32)
    out_ref[...] = jax.nn.sigmoid(z)


def _tc_decode(encoded):
    grid = (pl.cdiv(N, DM), pl.cdiv(N, DN))
    return pl.pallas_call(
        _decode_body,
        grid=grid,
        in_specs=[
            pl.BlockSpec((DM, C), lambda i, j: (i, 0)),
            pl.BlockSpec((DN, C), lambda i, j: (j, 0)),
        ],
        out_specs=pl.BlockSpec((DM, DN), lambda i, j: (i, j)),
        out_shape=jax.ShapeDtypeStruct((N, N), jnp.float32),
    )(encoded, encoded)


def kernel(x, edge_index, W_hidden, W_mean, W_logstd):
    src = edge_index[0].astype(jnp.int32)
    dst = edge_index[1].astype(jnp.int32)
    pad = EPAD - E
    src_p = jnp.concatenate([src, jnp.zeros((pad,), jnp.int32)]).reshape(NW, KCH, B)
    dst_p = jnp.concatenate([dst, jnp.full((pad,), N, jnp.int32)]).reshape(NW, KCH, B)

    onesW = jnp.ones((B, W), jnp.float32)
    zerosW = jnp.zeros((NP, W), jnp.float32)
    Wcat = jnp.concatenate([W_mean, W_logstd], axis=1)
    eps = jax.random.uniform(jax.random.key(42), (N, C), dtype=jnp.float32)

    h = _tc_prep0(x, W_hidden)
    deg_parts = _sc_deg(dst_p, onesW, zerosW)
    h1p, dinv = _tc_prep1(deg_parts, h)
    acc1 = _sc_spmm(src_p, dst_p, h1p, zerosW)
    g1p = _tc_prep2(acc1, h1p, dinv, Wcat)
    acc2 = _sc_spmm(src_p, dst_p, g1p, zerosW)
    encoded = _tc_encode(acc2, g1p, dinv, eps)
    return _tc_decode(encoded)


# speedup vs baseline: 8.7307x; 1.0185x over previous
import jax
import jax.numpy as jnp
from jax import lax
from jax.experimental import pallas as pl
from jax.experimental.pallas import tpu as pltpu
from jax.experimental.pallas import tpu_sc as plsc

N = 10000
E = 320000
D = 128
H = 32
C = 16

NC = 2
NS = 16
NW = NC * NS
B = 128
KCH = 80
EW = KCH * B
EPAD = NW * EW
NP = 10112
RT = NP // NS
W = 128


def _sc_mesh():
    return plsc.VectorSubcoreMesh(core_axis_name="c", subcore_axis_name="s")


def _deg_body(dst_hbm, ones_hbm, zeros_hbm, out_hbm, dst_v, ones_v, acc_sh, sem):
    cid = lax.axis_index("c")
    sid = lax.axis_index("s")
    wid = sid * NC + cid

    pltpu.sync_copy(zeros_hbm.at[pl.ds(sid * RT, RT)],
                    acc_sh.at[pl.ds(sid * RT, RT)])
    pltpu.sync_copy(dst_hbm.at[wid], dst_v)
    pltpu.sync_copy(ones_hbm, ones_v)
    plsc.subcore_barrier()

    def body(j, carry):
        pltpu.sync_copy(ones_v, acc_sh.at[dst_v.at[j]], add=True)
        return carry

    lax.fori_loop(0, KCH, body, 0)
    plsc.subcore_barrier()
    pltpu.sync_copy(acc_sh.at[pl.ds(sid * RT, RT)],
                    out_hbm.at[cid, pl.ds(sid * RT, RT)])


def _sc_deg(dst_p, onesW, zerosW):
    call = pl.kernel(
        _deg_body,
        out_type=jax.ShapeDtypeStruct((NC, NP, W), jnp.float32),
        mesh=_sc_mesh(),
        scratch_types=[
            pltpu.VMEM((KCH, B), jnp.int32),
            pltpu.VMEM((B, W), jnp.float32),
            pltpu.VMEM_SHARED((NP, W), jnp.float32),
            pltpu.SemaphoreType.DMA,
        ],
    )
    return call(dst_p, onesW, zerosW)


def _spmm_body(src_hbm, dst_hbm, feat_hbm, zeros_hbm, out_hbm,
               src_v, d0, d1, r0, r1, acc_sh, sd0, sd1, sg0, sg1):
    cid = lax.axis_index("c")
    sid = lax.axis_index("s")
    wid = sid * NC + cid

    pltpu.sync_copy(zeros_hbm.at[pl.ds(sid * RT, RT)],
                    acc_sh.at[pl.ds(sid * RT, RT)])
    pltpu.sync_copy(src_hbm.at[wid], src_v)
    plsc.subcore_barrier()

    def body(p, carry):
        j0 = 2 * p
        j1 = j0 + 1
        hd0 = pltpu.async_copy(dst_hbm.at[wid, j0], d0, sd0)
        hd1 = pltpu.async_copy(dst_hbm.at[wid, j1], d1, sd1)
        g0 = pltpu.async_copy(feat_hbm.at[src_v.at[j0]], r0, sg0)
        g1 = pltpu.async_copy(feat_hbm.at[src_v.at[j1]], r1, sg1)
        g0.wait()
        hd0.wait()
        pltpu.sync_copy(r0, acc_sh.at[d0], add=True)
        g1.wait()
        hd1.wait()
        pltpu.sync_copy(r1, acc_sh.at[d1], add=True)
        return carry

    lax.fori_loop(0, KCH // 2, body, 0)
    plsc.subcore_barrier()
    pltpu.sync_copy(acc_sh.at[pl.ds(sid * RT, RT)],
                    out_hbm.at[cid, pl.ds(sid * RT, RT)])


def _sc_spmm(src_p, dst_p, feat, zerosW):
    call = pl.kernel(
        _spmm_body,
        out_type=jax.ShapeDtypeStruct((NC, NP, W), jnp.float32),
        mesh=_sc_mesh(),
        scratch_types=[
            pltpu.VMEM((KCH, B), jnp.int32),
            pltpu.VMEM((B,), jnp.int32),
            pltpu.VMEM((B,), jnp.int32),
            pltpu.VMEM((B, W), jnp.float32),
            pltpu.VMEM((B, W), jnp.float32),
            pltpu.VMEM_SHARED((NP, W), jnp.float32),
            pltpu.SemaphoreType.DMA,
            pltpu.SemaphoreType.DMA,
            pltpu.SemaphoreType.DMA,
            pltpu.SemaphoreType.DMA,
        ],
    )
    return call(src_p, dst_p, feat, zerosW)


RBLK = 1000


def _prep0_body(x_ref, w_ref, h_ref):
    h_ref[...] = jnp.dot(x_ref[...], w_ref[...],
                         preferred_element_type=jnp.float32)


def _prep1_body(deg_ref, h_ref, h1_ref, dinv_ref):
    deg = deg_ref[0, :, 0:1] + deg_ref[1, :, 0:1] + 1.0
    dinv = lax.rsqrt(jnp.maximum(deg, 1.0))
    dinvH = jnp.broadcast_to(dinv, (RBLK, H))
    h1 = dinvH * h_ref[...]
    h1_ref[...] = jnp.concatenate(
        [h1, jnp.zeros((RBLK, W - H), jnp.float32)], axis=1)
    dinv_ref[...] = dinvH


def _prep2_body(acc_ref, h1_ref, dinv_ref, wcat_ref, g1_ref):
    s = acc_ref[0, :, :H] + acc_ref[1, :, :H] + h1_ref[:, :H]
    hidden = jnp.maximum(dinv_ref[...] * s, 0.0)
    g = jnp.dot(hidden, wcat_ref[...], preferred_element_type=jnp.float32)
    g1_ref[...] = jnp.concatenate(
        [dinv_ref[...] * g, jnp.zeros((RBLK, W - H), jnp.float32)], axis=1)


def _encode_body(acc_ref, g1_ref, dinv_ref, eps_ref, enc_ref):
    out2 = dinv_ref[...] * (acc_ref[0, :, :H] + acc_ref[1, :, :H]
                            + g1_ref[:, :H])
    means = out2[:, :C]
    log_std = out2[:, C:]
    enc_ref[...] = means + jnp.exp(log_std) * eps_ref[...]


def _tc_prep0(x, W_hidden):
    grid = (N // RBLK,)
    return pl.pallas_call(
        _prep0_body,
        grid=grid,
        in_specs=[
            pl.BlockSpec((RBLK, D), lambda i: (i, 0)),
            pl.BlockSpec((D, H), lambda i: (0, 0)),
        ],
        out_specs=pl.BlockSpec((RBLK, H), lambda i: (i, 0)),
        out_shape=jax.ShapeDtypeStruct((N, H), jnp.float32),
    )(x, W_hidden)


def _tc_prep1(deg_parts, h):
    grid = (N // RBLK,)
    return pl.pallas_call(
        _prep1_body,
        grid=grid,
        in_specs=[
            pl.BlockSpec((NC, RBLK, W), lambda i: (0, i, 0)),
            pl.BlockSpec((RBLK, H), lambda i: (i, 0)),
        ],
        out_specs=[
            pl.BlockSpec((RBLK, W), lambda i: (i, 0)),
            pl.BlockSpec((RBLK, H), lambda i: (i, 0)),
        ],
        out_shape=[
            jax.ShapeDtypeStruct((N, W), jnp.float32),
            jax.ShapeDtypeStruct((N, H), jnp.float32),
        ],
    )(deg_parts, h)


def _tc_prep2(acc_parts, h1p, dinv, Wcat):
    grid = (N // RBLK,)
    return pl.pallas_call(
        _prep2_body,
        grid=grid,
        in_specs=[
            pl.BlockSpec((NC, RBLK, W), lambda i: (0, i, 0)),
            pl.BlockSpec((RBLK, W), lambda i: (i, 0)),
            pl.BlockSpec((RBLK, H), lambda i: (i, 0)),
            pl.BlockSpec((H, H), lambda i: (0, 0)),
        ],
        out_specs=pl.BlockSpec((RBLK, W), lambda i: (i, 0)),
        out_shape=jax.ShapeDtypeStruct((N, W), jnp.float32),
    )(acc_parts, h1p, dinv, Wcat)


def _tc_encode(acc_parts, g1p, dinv, eps):
    grid = (N // RBLK,)
    return pl.pallas_call(
        _encode_body,
        grid=grid,
        in_specs=[
            pl.BlockSpec((NC, RBLK, W), lambda i: (0, i, 0)),
            pl.BlockSpec((RBLK, W), lambda i: (i, 0)),
            pl.BlockSpec((RBLK, H), lambda i: (i, 0)),
            pl.BlockSpec((RBLK, C), lambda i: (i, 0)),
        ],
        out_specs=pl.BlockSpec((RBLK, C), lambda i: (i, 0)),
        out_shape=jax.ShapeDtypeStruct((N, C), jnp.float32),
    )(acc_parts, g1p, dinv, eps)


DM = 512
DN = 2048


def _decode_body(a_ref, b_ref, out_ref):
    z = lax.dot_general(a_ref[...], b_ref[...],
                        (((1,), (1,)), ((), ())),
                        preferred_element_type=jnp.float32)
    out_ref[...] = jax.nn.sigmoid(z)


def _tc_decode(encoded):
    grid = (pl.cdiv(N, DM), pl.cdiv(N, DN))
    return pl.pallas_call(
        _decode_body,
        grid=grid,
        in_specs=[
            pl.BlockSpec((DM, C), lambda i, j: (i, 0)),
            pl.BlockSpec((DN, C), lambda i, j: (j, 0)),
        ],
        out_specs=pl.BlockSpec((DM, DN), lambda i, j: (i, j)),
        out_shape=jax.ShapeDtypeStruct((N, N), jnp.float32),
    )(encoded, encoded)


def kernel(x, edge_index, W_hidden, W_mean, W_logstd):
    src = edge_index[0].astype(jnp.int32)
    dst = edge_index[1].astype(jnp.int32)
    pad = EPAD - E
    src_p = jnp.concatenate([src, jnp.zeros((pad,), jnp.int32)]).reshape(NW, KCH, B)
    dst_p = jnp.concatenate([dst, jnp.full((pad,), N, jnp.int32)]).reshape(NW, KCH, B)

    onesW = jnp.ones((B, W), jnp.float32)
    zerosW = jnp.zeros((NP, W), jnp.float32)
    Wcat = jnp.concatenate([W_mean, W_logstd], axis=1)
    eps = jax.random.uniform(jax.random.key(42), (N, C), dtype=jnp.float32)

    h = _tc_prep0(x, W_hidden)
    deg_parts = _sc_deg(dst_p, onesW, zerosW)
    h1p, dinv = _tc_prep1(deg_parts, h)
    acc1 = _sc_spmm(src_p, dst_p, h1p, zerosW)
    g1p = _tc_prep2(acc1, h1p, dinv, Wcat)
    acc2 = _sc_spmm(src_p, dst_p, g1p, zerosW)
    encoded = _tc_encode(acc2, g1p, dinv, eps)
    return _tc_decode(encoded)
